# baseline (speedup 1.0000x reference)
import jax
import jax.numpy as jnp
from jax.experimental import pallas as pl

_N = 50000
_D = 256
_M = 512
_B = 1000
_G = _N // _B


def _mlp_segsum_kernel(x_ref, w1_ref, b1_ref, w2_ref, b2_ref, w3_ref, b3_ref,
                       ids_ref, out_ref):
    i = pl.program_id(0)
    x = x_ref[...].astype(jnp.bfloat16)
    h = jnp.dot(x, w1_ref[...].astype(jnp.bfloat16),
                preferred_element_type=jnp.float32) + b1_ref[...]
    h = h * jax.nn.sigmoid(h)
    h = jnp.dot(h.astype(jnp.bfloat16), w2_ref[...].astype(jnp.bfloat16),
                preferred_element_type=jnp.float32) + b2_ref[...]
    h = h * jax.nn.sigmoid(h)
    s = jnp.sum(h * w3_ref[...], axis=1, keepdims=True) + b3_ref[0, 0]

    ids = ids_ref[0, 0, :]
    seg = jax.lax.broadcasted_iota(jnp.int32, (_B, _M), 1)
    hit = ids[:, None] == seg
    partial = jnp.sum(jnp.where(hit, s, 0.0), axis=0, keepdims=True)

    @pl.when(i == 0)
    def _():
        out_ref[...] = jnp.zeros_like(out_ref)

    out_ref[...] += partial


def kernel(energy, W1, b1, W2, b2, W3, b3, batch):
    ids3 = batch.astype(jnp.int32).reshape(_G, 1, _B)
    out = pl.pallas_call(
        _mlp_segsum_kernel,
        grid=(_G,),
        in_specs=[
            pl.BlockSpec((_B, _D), lambda i: (i, 0)),
            pl.BlockSpec((_D, _D), lambda i: (0, 0)),
            pl.BlockSpec((1, _D), lambda i: (0, 0)),
            pl.BlockSpec((_D, _D), lambda i: (0, 0)),
            pl.BlockSpec((1, _D), lambda i: (0, 0)),
            pl.BlockSpec((1, _D), lambda i: (0, 0)),
            pl.BlockSpec((1, 1), lambda i: (0, 0)),
            pl.BlockSpec((1, 1, _B), lambda i: (i, 0, 0)),
        ],
        out_specs=pl.BlockSpec((1, _M), lambda i: (0, 0)),
        out_shape=jax.ShapeDtypeStruct((1, _M), jnp.float32),
    )(energy, W1, b1.reshape(1, _D), W2, b2.reshape(1, _D),
      W3.reshape(1, _D), b3.reshape(1, 1), ids3)
    return out[0]


# speedup vs baseline: 1.0421x; 1.0421x over previous
import jax
import jax.numpy as jnp
from jax.experimental import pallas as pl

_N = 50000
_D = 256
_M = 512
_B = 1000
_G = _N // _B


def _mlp_segsum_kernel(x_ref, w1_ref, b1_ref, w2_ref, b2_ref, w3_ref, b3_ref,
                       ids_ref, out_ref):
    i = pl.program_id(0)
    x = x_ref[...].astype(jnp.bfloat16)
    u = (jnp.dot(x, w1_ref[...], preferred_element_type=jnp.float32)
         + b1_ref[...]).astype(jnp.bfloat16)
    t = jnp.tanh(u)
    g = u + u * t
    u = (jnp.dot(g, w2_ref[...], preferred_element_type=jnp.float32)
         + b2_ref[...]).astype(jnp.bfloat16)
    t = jnp.tanh(u)
    h = (u + u * t).astype(jnp.float32)
    s = jnp.sum(h * w3_ref[...], axis=1, keepdims=True) + b3_ref[0, 0]

    ids = ids_ref[0, 0, :]
    seg = jax.lax.broadcasted_iota(jnp.int32, (_B, _M), 1)
    hit = ids[:, None] == seg
    partial = jnp.sum(jnp.where(hit, s, 0.0), axis=0, keepdims=True)

    @pl.when(i == 0)
    def _():
        out_ref[...] = jnp.zeros_like(out_ref)

    out_ref[...] += partial


def kernel(energy, W1, b1, W2, b2, W3, b3, batch):
    ids3 = batch.astype(jnp.int32).reshape(_G, 1, _B)
    w1h = (W1 * 0.5).astype(jnp.bfloat16)
    b1h = (b1 * 0.5).reshape(1, _D)
    w2h = (W2 * 0.5).astype(jnp.bfloat16)
    b2h = (b2 * 0.5).reshape(1, _D)
    out = pl.pallas_call(
        _mlp_segsum_kernel,
        grid=(_G,),
        in_specs=[
            pl.BlockSpec((_B, _D), lambda i: (i, 0)),
            pl.BlockSpec((_D, _D), lambda i: (0, 0)),
            pl.BlockSpec((1, _D), lambda i: (0, 0)),
            pl.BlockSpec((_D, _D), lambda i: (0, 0)),
            pl.BlockSpec((1, _D), lambda i: (0, 0)),
            pl.BlockSpec((1, _D), lambda i: (0, 0)),
            pl.BlockSpec((1, 1), lambda i: (0, 0)),
            pl.BlockSpec((1, 1, _B), lambda i: (i, 0, 0)),
        ],
        out_specs=pl.BlockSpec((1, _M), lambda i: (0, 0)),
        out_shape=jax.ShapeDtypeStruct((1, _M), jnp.float32),
    )(energy, w1h, b1h, w2h, b2h,
      W3.reshape(1, _D), b3.reshape(1, 1), ids3)
    return out[0]


# speedup vs baseline: 1.3615x; 1.3065x over previous
import jax
import jax.numpy as jnp
from jax.experimental import pallas as pl

_N = 50000
_D = 256
_M = 512
_B = 1000
_SPLIT = 2
_G = _N // (_B * _SPLIT)


def _mlp_segsum_kernel(*refs):
    x_refs = refs[:_SPLIT]
    w1_ref, b1_ref, w2_ref, b2_ref, w3_ref, b3_ref = refs[_SPLIT:_SPLIT + 6]
    ids_refs = refs[_SPLIT + 6:2 * _SPLIT + 6]
    out_ref = refs[-1]
    i = pl.program_id(0)

    partial = jnp.zeros((1, _M), dtype=jnp.float32)
    for x_ref, ids_ref in zip(x_refs, ids_refs):
        x = x_ref[...].astype(jnp.bfloat16)
        u = (jnp.dot(x, w1_ref[...], preferred_element_type=jnp.float32)
             + b1_ref[...]).astype(jnp.bfloat16)
        t = jnp.tanh(u)
        g = u + u * t
        u = (jnp.dot(g, w2_ref[...], preferred_element_type=jnp.float32)
             + b2_ref[...]).astype(jnp.bfloat16)
        t = jnp.tanh(u)
        h = (u + u * t).astype(jnp.float32)
        s = jnp.sum(h * w3_ref[...], axis=1, keepdims=True) + b3_ref[0, 0]

        ids = ids_ref[0, 0, :]
        seg = jax.lax.broadcasted_iota(jnp.int32, (_B, _M), 1)
        hit = ids[:, None] == seg
        partial = partial + jnp.sum(jnp.where(hit, s, 0.0), axis=0,
                                    keepdims=True)

    @pl.when(i == 0)
    def _():
        out_ref[...] = jnp.zeros_like(out_ref)

    out_ref[...] += partial


def _x_spec(k):
    return pl.BlockSpec((_B, _D), lambda i, k=k: (_SPLIT * i + k, 0))


def _ids_spec(k):
    return pl.BlockSpec((1, 1, _B), lambda i, k=k: (_SPLIT * i + k, 0, 0))


def kernel(energy, W1, b1, W2, b2, W3, b3, batch):
    ids3 = batch.astype(jnp.int32).reshape(_N // _B, 1, _B)
    w1h = (W1 * 0.5).astype(jnp.bfloat16)
    b1h = (b1 * 0.5).reshape(1, _D)
    w2h = (W2 * 0.5).astype(jnp.bfloat16)
    b2h = (b2 * 0.5).reshape(1, _D)
    out = pl.pallas_call(
        _mlp_segsum_kernel,
        grid=(_G,),
        in_specs=(
            [_x_spec(k) for k in range(_SPLIT)]
            + [
                pl.BlockSpec((_D, _D), lambda i: (0, 0)),
                pl.BlockSpec((1, _D), lambda i: (0, 0)),
                pl.BlockSpec((_D, _D), lambda i: (0, 0)),
                pl.BlockSpec((1, _D), lambda i: (0, 0)),
                pl.BlockSpec((1, _D), lambda i: (0, 0)),
                pl.BlockSpec((1, 1), lambda i: (0, 0)),
            ]
            + [_ids_spec(k) for k in range(_SPLIT)]
        ),
        out_specs=pl.BlockSpec((1, _M), lambda i: (0, 0)),
        out_shape=jax.ShapeDtypeStruct((1, _M), jnp.float32),
    )(*([energy] * _SPLIT
        + [w1h, b1h, w2h, b2h, W3.reshape(1, _D), b3.reshape(1, 1)]
        + [ids3] * _SPLIT))
    return out[0]
